# Initial kernel scaffold; baseline (speedup 1.0000x reference)
import jax
import jax.numpy as jnp
from jax.experimental import pallas as pl


def kernel(x_p, x_d, edge_attr_p, edge_attr_d, edge_index_p, edge_index_d, x_p_batch, x_d_batch, Wp1, bp1, Wp2, bp2, Wp3, bp3, Wd1, bd1, Wd2, bd2, Wd3, bd3, Wl1, bl1, Wl2, bl2):
    raise NotImplementedError("write your pallas kernel here")


# speedup vs baseline: 7.1418x; 7.1418x over previous
import functools

import jax
import jax.numpy as jnp
from jax import lax
from jax.experimental import pallas as pl
from jax.experimental.pallas import tpu as pltpu
from jax.experimental.pallas import tpu_sc as plsc

N = 10000
E = 320000
D = 128
H = 96
G = 64

NP = 10240
HP = 128
NTILE = 16
NCORE = 2
C = 128
NCH = 160
KI = 8
NGRP = NCH // KI
TE = NCH * C
EPAD = NTILE * TE
RPT = NP // NTILE
RB = 1024

_F32 = jnp.float32
_HIGH = lax.Precision.HIGHEST


def _deg_body(dst_hbm, deg_p, deg_d, didx, onesv, acc):
    c = lax.axis_index("c")
    s = lax.axis_index("s")

    one16 = jnp.ones((16,), _F32)

    def _fill(i, carry):
        for k in range(HP // 16):
            onesv[i, pl.ds(k * 16, 16)] = one16
        return carry

    lax.fori_loop(0, C, _fill, 0)

    r0 = s * RPT
    for t in range(RPT // C):
        pltpu.sync_copy(onesv, acc.at[pl.ds(r0 + t * C, C)])
    plsc.subcore_barrier()

    def _group(g, carry):
        pltpu.sync_copy(dst_hbm.at[c, s, pl.ds(g * KI, KI)], didx)
        for k in range(KI):
            pltpu.sync_copy(onesv, acc.at[didx.at[k]], add=True)
        return carry

    lax.fori_loop(0, NGRP, _group, 0)

    plsc.subcore_barrier()

    @pl.when(c == 0)
    def _():
        pltpu.sync_copy(acc.at[pl.ds(r0, RPT)], deg_p.at[pl.ds(r0, RPT)])

    @pl.when(c == 1)
    def _():
        pltpu.sync_copy(acc.at[pl.ds(r0, RPT)], deg_d.at[pl.ds(r0, RPT)])


@functools.cache
def _deg_kernel():
    mesh = plsc.VectorSubcoreMesh(core_axis_name="c", subcore_axis_name="s",
                                  num_cores=NCORE, num_subcores=NTILE)
    return pl.kernel(
        _deg_body,
        out_type=(
            jax.ShapeDtypeStruct((NP, HP), _F32),
            jax.ShapeDtypeStruct((NP, HP), _F32),
        ),
        mesh=mesh,
        scratch_types=[
            pltpu.VMEM((KI, C), jnp.int32),
            pltpu.VMEM((C, HP), _F32),
            pltpu.VMEM_SHARED((NP, HP), _F32),
        ],
    )


def _scatter_one_branch(y, s_out, src_hbm, dst_hbm, c, s,
                        sidx, didx, rows, semi, semr, acc):
    r0 = s * RPT
    pltpu.sync_copy(y.at[pl.ds(r0, RPT)], acc.at[pl.ds(r0, RPT)])

    def idx_src(g):
        return src_hbm.at[c, s, pl.ds(g * KI, KI)]

    def idx_dst(g):
        return dst_hbm.at[c, s, pl.ds(g * KI, KI)]

    plsc.subcore_barrier()

    def _group(g, carry):
        pltpu.sync_copy(idx_src(g), sidx.at[0])
        pltpu.sync_copy(idx_dst(g), didx.at[0])
        for k in range(KI):
            pltpu.async_copy(y.at[sidx.at[0, k]], rows[0], semr[0]).wait()
            pltpu.sync_copy(rows[0], acc.at[didx.at[0, k]], add=True)
        return carry

    lax.fori_loop(0, NGRP, _group, 0)

    plsc.subcore_barrier()
    pltpu.sync_copy(acc.at[pl.ds(r0, RPT)], s_out.at[pl.ds(r0, RPT)])


def _scatter_body(y_p, y_d, src_hbm, dst_hbm, s_p, s_d,
                  sidx, didx, rows0, rows1, semi0, semi1, semr0, semr1, acc):
    c = lax.axis_index("c")
    s = lax.axis_index("s")
    rows = (rows0, rows1)
    semi = (semi0, semi1)
    semr = (semr0, semr1)

    @pl.when(c == 0)
    def _():
        _scatter_one_branch(y_p, s_p, src_hbm, dst_hbm, c, s,
                            sidx, didx, rows, semi, semr, acc)

    @pl.when(c == 1)
    def _():
        _scatter_one_branch(y_d, s_d, src_hbm, dst_hbm, c, s,
                            sidx, didx, rows, semi, semr, acc)


@functools.cache
def _scatter_kernel():
    mesh = plsc.VectorSubcoreMesh(core_axis_name="c", subcore_axis_name="s",
                                  num_cores=NCORE, num_subcores=NTILE)
    return pl.kernel(
        _scatter_body,
        out_type=(
            jax.ShapeDtypeStruct((NP, HP), _F32),
            jax.ShapeDtypeStruct((NP, HP), _F32),
        ),
        mesh=mesh,
        scratch_types=[
            pltpu.VMEM((2, KI, C), jnp.int32),
            pltpu.VMEM((2, KI, C), jnp.int32),
            pltpu.VMEM((C, HP), _F32),
            pltpu.VMEM((C, HP), _F32),
            pltpu.SemaphoreType.DMA,
            pltpu.SemaphoreType.DMA,
            pltpu.SemaphoreType.DMA,
            pltpu.SemaphoreType.DMA,
            pltpu.VMEM_SHARED((NP, HP), _F32),
        ],
    )


def _t0_body(degp_ref, degd_ref, dinv_ref):
    dp = degp_ref[...][:, :1]
    dd = degd_ref[...][:, :1]
    dinv_ref[...] = 1.0 / jnp.sqrt(jnp.stack([dp, dd]))


_t0 = pl.pallas_call(
    _t0_body,
    grid=(NP // RB,),
    in_specs=[
        pl.BlockSpec((RB, HP), lambda i: (i, 0)),
        pl.BlockSpec((RB, HP), lambda i: (i, 0)),
    ],
    out_specs=pl.BlockSpec((NCORE, RB, 1), lambda i: (0, i, 0)),
    out_shape=jax.ShapeDtypeStruct((NCORE, NP, 1), _F32),
)


def _t1_body(x_ref, w_ref, dinv_ref, y_ref):
    y_ref[...] = jnp.dot(x_ref[...], w_ref[...]) * dinv_ref[...]


_t1 = pl.pallas_call(
    _t1_body,
    grid=(NP // RB,),
    in_specs=[
        pl.BlockSpec((RB, D), lambda i: (i, 0)),
        pl.BlockSpec((D, HP), lambda i: (0, 0)),
        pl.BlockSpec((RB, 1), lambda i: (i, 0)),
    ],
    out_specs=pl.BlockSpec((RB, HP), lambda i: (i, 0)),
    out_shape=jax.ShapeDtypeStruct((NP, HP), _F32),
)


def _t2_body(s_ref, dinv_ref, b_ref, w_ref, y_ref):
    h = jax.nn.relu(s_ref[...] * dinv_ref[...] + b_ref[...][None, :])
    y_ref[...] = jnp.dot(h, w_ref[...]) * dinv_ref[...]


_t2 = pl.pallas_call(
    _t2_body,
    grid=(NP // RB,),
    in_specs=[
        pl.BlockSpec((RB, HP), lambda i: (i, 0)),
        pl.BlockSpec((RB, 1), lambda i: (i, 0)),
        pl.BlockSpec((HP,), lambda i: (0,)),
        pl.BlockSpec((HP, HP), lambda i: (0, 0)),
    ],
    out_specs=pl.BlockSpec((RB, HP), lambda i: (i, 0)),
    out_shape=jax.ShapeDtypeStruct((NP, HP), _F32),
)


def _t3_body(s_ref, dinv_ref, b_ref, batch_ref, psum_ref, cnt_ref):
    i = pl.program_id(0)
    h3 = s_ref[...] * dinv_ref[...] + b_ref[...][None, :]
    bt = batch_ref[...][:, 0]
    oneh = (bt[None, :] == lax.broadcasted_iota(jnp.int32, (G, RB), 0))
    oneh = oneh.astype(_F32)
    ps = jnp.dot(oneh, h3, precision=_HIGH)
    cs = jnp.sum(oneh, axis=1, keepdims=True)

    @pl.when(i == 0)
    def _():
        psum_ref[...] = ps
        cnt_ref[...] = cs

    @pl.when(i > 0)
    def _():
        psum_ref[...] += ps
        cnt_ref[...] += cs


_t3 = pl.pallas_call(
    _t3_body,
    grid=(NP // RB,),
    in_specs=[
        pl.BlockSpec((RB, HP), lambda i: (i, 0)),
        pl.BlockSpec((RB, 1), lambda i: (i, 0)),
        pl.BlockSpec((HP,), lambda i: (0,)),
        pl.BlockSpec((RB, 1), lambda i: (i, 0)),
    ],
    out_specs=(
        pl.BlockSpec((G, HP), lambda i: (0, 0)),
        pl.BlockSpec((G, 1), lambda i: (0, 0)),
    ),
    out_shape=(
        jax.ShapeDtypeStruct((G, HP), _F32),
        jax.ShapeDtypeStruct((G, 1), _F32),
    ),
)


def _t4_body(psp_ref, cp_ref, psd_ref, cd_ref, w1_ref, b1_ref, w2_ref, b2_ref,
             out_ref):
    pp = psp_ref[...][:, :H] / jnp.maximum(cp_ref[...], 1.0)
    pd = psd_ref[...][:, :H] / jnp.maximum(cd_ref[...], 1.0)
    xc = jnp.concatenate([pp, pd], axis=1)
    z = jax.nn.relu(jnp.dot(xc, w1_ref[...]) + b1_ref[...][None, :])
    out_ref[...] = jnp.dot(z, w2_ref[...]) + b2_ref[0]


_t4 = pl.pallas_call(
    _t4_body,
    in_specs=[
        pl.BlockSpec((G, HP), lambda: (0, 0)),
        pl.BlockSpec((G, 1), lambda: (0, 0)),
        pl.BlockSpec((G, HP), lambda: (0, 0)),
        pl.BlockSpec((G, 1), lambda: (0, 0)),
        pl.BlockSpec((2 * H, H), lambda: (0, 0)),
        pl.BlockSpec((H,), lambda: (0,)),
        pl.BlockSpec((H, 1), lambda: (0, 0)),
        pl.BlockSpec(memory_space=pltpu.SMEM),
    ],
    out_specs=pl.BlockSpec((G, 1), lambda: (0, 0)),
    out_shape=jax.ShapeDtypeStruct((G, 1), _F32),
)


def _prep_edges(edge_index):
    pad = jnp.full((EPAD - E,), N, jnp.int32)
    src = jnp.concatenate([edge_index[0], pad]).reshape(NTILE, NCH, C)
    dst = jnp.concatenate([edge_index[1], pad]).reshape(NTILE, NCH, C)
    return src, dst


def _pad_rows(x):
    return jnp.concatenate([x, jnp.zeros((NP - N, x.shape[1]), x.dtype)])


def _pad_w(w):
    return jnp.pad(w, ((0, 0), (0, HP - w.shape[1])))


def _pad_b(b):
    return jnp.pad(b, (0, HP - b.shape[0]))


def _pad_w2(w):
    return jnp.pad(w, ((0, HP - w.shape[0]), (0, HP - w.shape[1])))


def kernel(x_p, x_d, edge_attr_p, edge_attr_d, edge_index_p, edge_index_d,
           x_p_batch, x_d_batch,
           Wp1, bp1, Wp2, bp2, Wp3, bp3,
           Wd1, bd1, Wd2, bd2, Wd3, bd3,
           Wl1, bl1, Wl2, bl2):
    del edge_attr_p, edge_attr_d

    src_p, dst_p = _prep_edges(edge_index_p)
    src_d, dst_d = _prep_edges(edge_index_d)
    src_all = jnp.stack([src_p, src_d])
    dst_all = jnp.stack([dst_p, dst_d])

    xp = _pad_rows(x_p)
    xd = _pad_rows(x_d)
    bpad = jnp.full((NP - N,), G, jnp.int32)
    batch_p = jnp.concatenate([x_p_batch, bpad]).reshape(NP, 1)
    batch_d = jnp.concatenate([x_d_batch, bpad]).reshape(NP, 1)

    deg_p, deg_d = _deg_kernel()(dst_all)
    dinv = _t0(deg_p, deg_d)
    dinv_p, dinv_d = dinv[0], dinv[1]

    y1p = _t1(xp, _pad_w(Wp1), dinv_p)
    y1d = _t1(xd, _pad_w(Wd1), dinv_d)
    s1p, s1d = _scatter_kernel()(y1p, y1d, src_all, dst_all)

    y2p = _t2(s1p, dinv_p, _pad_b(bp1), _pad_w2(Wp2))
    y2d = _t2(s1d, dinv_d, _pad_b(bd1), _pad_w2(Wd2))
    s2p, s2d = _scatter_kernel()(y2p, y2d, src_all, dst_all)

    y3p = _t2(s2p, dinv_p, _pad_b(bp2), _pad_w2(Wp3))
    y3d = _t2(s2d, dinv_d, _pad_b(bd2), _pad_w2(Wd3))
    s3p, s3d = _scatter_kernel()(y3p, y3d, src_all, dst_all)

    psp, cp = _t3(s3p, dinv_p, _pad_b(bp3), batch_p)
    psd, cd = _t3(s3d, dinv_d, _pad_b(bd3), batch_d)

    return _t4(psp, cp, psd, cd, Wl1, bl1, Wl2, bl2)


# speedup vs baseline: 8.5796x; 1.2013x over previous
import functools

import jax
import jax.numpy as jnp
from jax import lax
from jax.experimental import pallas as pl
from jax.experimental.pallas import tpu as pltpu
from jax.experimental.pallas import tpu_sc as plsc

N = 10000
E = 320000
D = 128
H = 96
G = 64

NP = 10240
HP = 128
NTILE = 16
NCORE = 2
C = 128
NCH = 160
KI = 16
NGRP = NCH // KI
TE = NCH * C
EPAD = NTILE * TE
RPT = NP // NTILE
RB = 1024

_F32 = jnp.float32
_HIGH = lax.Precision.HIGHEST


def _deg_body(dst_hbm, deg_p, deg_d, didx, onesv, acc):
    c = lax.axis_index("c")
    s = lax.axis_index("s")

    one16 = jnp.ones((16,), _F32)

    def _fill(i, carry):
        for k in range(HP // 16):
            onesv[i, pl.ds(k * 16, 16)] = one16
        return carry

    lax.fori_loop(0, C, _fill, 0)

    r0 = s * RPT
    for t in range(RPT // C):
        pltpu.sync_copy(onesv, acc.at[pl.ds(r0 + t * C, C)])
    plsc.subcore_barrier()

    def _group(g, carry):
        pltpu.sync_copy(dst_hbm.at[c, s, pl.ds(g * KI, KI)], didx)
        for k in range(KI):
            pltpu.sync_copy(onesv, acc.at[didx.at[k]], add=True)
        return carry

    lax.fori_loop(0, NGRP, _group, 0)

    plsc.subcore_barrier()

    @pl.when(c == 0)
    def _():
        pltpu.sync_copy(acc.at[pl.ds(r0, RPT)], deg_p.at[pl.ds(r0, RPT)])

    @pl.when(c == 1)
    def _():
        pltpu.sync_copy(acc.at[pl.ds(r0, RPT)], deg_d.at[pl.ds(r0, RPT)])


@functools.cache
def _deg_kernel():
    mesh = plsc.VectorSubcoreMesh(core_axis_name="c", subcore_axis_name="s",
                                  num_cores=NCORE, num_subcores=NTILE)
    return pl.kernel(
        _deg_body,
        out_type=(
            jax.ShapeDtypeStruct((NP, HP), _F32),
            jax.ShapeDtypeStruct((NP, HP), _F32),
        ),
        mesh=mesh,
        scratch_types=[
            pltpu.VMEM((KI, C), jnp.int32),
            pltpu.VMEM((C, HP), _F32),
            pltpu.VMEM_SHARED((NP, HP), _F32),
        ],
    )


def _scatter_one_branch(y, s_out, src_hbm, dst_hbm, c, s,
                        sidx, didx, rows, semi, semr, semsc, acc):
    r0 = s * RPT
    pltpu.sync_copy(y.at[pl.ds(r0, RPT)], acc.at[pl.ds(r0, RPT)])

    def idx_src(g):
        return src_hbm.at[c, s, pl.ds(g * KI, KI)]

    def idx_dst(g):
        return dst_hbm.at[c, s, pl.ds(g * KI, KI)]

    pltpu.async_copy(idx_src(0), sidx.at[0], semi[0])
    pltpu.async_copy(idx_dst(0), didx.at[0], semi[0])
    plsc.subcore_barrier()
    pltpu.make_async_copy(idx_src(0), sidx.at[0], semi[0]).wait()
    pltpu.make_async_copy(idx_dst(0), didx.at[0], semi[0]).wait()
    pltpu.async_copy(y.at[sidx.at[0, 0]], rows[0], semr[0])
    pltpu.async_copy(y.at[sidx.at[0, 1]], rows[1], semr[1])

    def _pair(m, carry):
        for half in range(2):
            g = 2 * m + half
            sl, nsl = half, 1 - half

            @pl.when(g + 1 < NGRP)
            def _():
                pltpu.async_copy(idx_src(g + 1), sidx.at[nsl], semi[nsl])
                pltpu.async_copy(idx_dst(g + 1), didx.at[nsl], semi[nsl])

            for k in range(KI):
                rb = k % 2
                pltpu.make_async_copy(y.at[sidx.at[sl, k]], rows[rb],
                                      semr[rb]).wait()
                pltpu.async_copy(rows[rb], acc.at[didx.at[sl, k]],
                                 semsc[rb], add=True)
                pltpu.make_async_copy(rows[rb], acc.at[didx.at[sl, k]],
                                      semsc[rb]).wait()
                if k < KI - 2:
                    pltpu.async_copy(y.at[sidx.at[sl, k + 2]], rows[rb],
                                     semr[rb])
                elif k == KI - 2:

                    @pl.when(g + 1 < NGRP)
                    def _():
                        pltpu.make_async_copy(idx_src(g + 1), sidx.at[nsl],
                                              semi[nsl]).wait()
                        pltpu.make_async_copy(idx_dst(g + 1), didx.at[nsl],
                                              semi[nsl]).wait()
                        pltpu.async_copy(y.at[sidx.at[nsl, 0]], rows[rb],
                                         semr[rb])
                else:

                    @pl.when(g + 1 < NGRP)
                    def _():
                        pltpu.async_copy(y.at[sidx.at[nsl, 1]], rows[rb],
                                         semr[rb])
        return carry

    lax.fori_loop(0, NGRP // 2, _pair, 0)

    plsc.subcore_barrier()
    pltpu.sync_copy(acc.at[pl.ds(r0, RPT)], s_out.at[pl.ds(r0, RPT)])


def _scatter_body(y_p, y_d, src_hbm, dst_hbm, s_p, s_d,
                  sidx, didx, rows0, rows1, semi0, semi1, semr0, semr1,
                  semsc0, semsc1, acc):
    c = lax.axis_index("c")
    s = lax.axis_index("s")
    rows = (rows0, rows1)
    semi = (semi0, semi1)
    semr = (semr0, semr1)
    semsc = (semsc0, semsc1)

    @pl.when(c == 0)
    def _():
        _scatter_one_branch(y_p, s_p, src_hbm, dst_hbm, c, s,
                            sidx, didx, rows, semi, semr, semsc, acc)

    @pl.when(c == 1)
    def _():
        _scatter_one_branch(y_d, s_d, src_hbm, dst_hbm, c, s,
                            sidx, didx, rows, semi, semr, semsc, acc)


@functools.cache
def _scatter_kernel():
    mesh = plsc.VectorSubcoreMesh(core_axis_name="c", subcore_axis_name="s",
                                  num_cores=NCORE, num_subcores=NTILE)
    return pl.kernel(
        _scatter_body,
        out_type=(
            jax.ShapeDtypeStruct((NP, HP), _F32),
            jax.ShapeDtypeStruct((NP, HP), _F32),
        ),
        mesh=mesh,
        scratch_types=[
            pltpu.VMEM((2, KI, C), jnp.int32),
            pltpu.VMEM((2, KI, C), jnp.int32),
            pltpu.VMEM((C, HP), _F32),
            pltpu.VMEM((C, HP), _F32),
            pltpu.SemaphoreType.DMA,
            pltpu.SemaphoreType.DMA,
            pltpu.SemaphoreType.DMA,
            pltpu.SemaphoreType.DMA,
            pltpu.SemaphoreType.DMA,
            pltpu.SemaphoreType.DMA,
            pltpu.VMEM_SHARED((NP, HP), _F32),
        ],
    )


def _t0_body(degp_ref, degd_ref, dinv_ref):
    dp = degp_ref[...][:, :1]
    dd = degd_ref[...][:, :1]
    dinv_ref[...] = 1.0 / jnp.sqrt(jnp.stack([dp, dd]))


_t0 = pl.pallas_call(
    _t0_body,
    grid=(NP // RB,),
    in_specs=[
        pl.BlockSpec((RB, HP), lambda i: (i, 0)),
        pl.BlockSpec((RB, HP), lambda i: (i, 0)),
    ],
    out_specs=pl.BlockSpec((NCORE, RB, 1), lambda i: (0, i, 0)),
    out_shape=jax.ShapeDtypeStruct((NCORE, NP, 1), _F32),
)


def _t1_body(x_ref, w_ref, dinv_ref, y_ref):
    y_ref[...] = jnp.dot(x_ref[...], w_ref[...]) * dinv_ref[...]


_t1 = pl.pallas_call(
    _t1_body,
    grid=(NP // RB,),
    in_specs=[
        pl.BlockSpec((RB, D), lambda i: (i, 0)),
        pl.BlockSpec((D, HP), lambda i: (0, 0)),
        pl.BlockSpec((RB, 1), lambda i: (i, 0)),
    ],
    out_specs=pl.BlockSpec((RB, HP), lambda i: (i, 0)),
    out_shape=jax.ShapeDtypeStruct((NP, HP), _F32),
)


def _t2_body(s_ref, dinv_ref, b_ref, w_ref, y_ref):
    h = jax.nn.relu(s_ref[...] * dinv_ref[...] + b_ref[...][None, :])
    y_ref[...] = jnp.dot(h, w_ref[...]) * dinv_ref[...]


_t2 = pl.pallas_call(
    _t2_body,
    grid=(NP // RB,),
    in_specs=[
        pl.BlockSpec((RB, HP), lambda i: (i, 0)),
        pl.BlockSpec((RB, 1), lambda i: (i, 0)),
        pl.BlockSpec((HP,), lambda i: (0,)),
        pl.BlockSpec((HP, HP), lambda i: (0, 0)),
    ],
    out_specs=pl.BlockSpec((RB, HP), lambda i: (i, 0)),
    out_shape=jax.ShapeDtypeStruct((NP, HP), _F32),
)


def _t3_body(s_ref, dinv_ref, b_ref, batch_ref, psum_ref, cnt_ref):
    i = pl.program_id(0)
    h3 = s_ref[...] * dinv_ref[...] + b_ref[...][None, :]
    bt = batch_ref[...][:, 0]
    oneh = (bt[None, :] == lax.broadcasted_iota(jnp.int32, (G, RB), 0))
    oneh = oneh.astype(_F32)
    ps = jnp.dot(oneh, h3, precision=_HIGH)
    cs = jnp.sum(oneh, axis=1, keepdims=True)

    @pl.when(i == 0)
    def _():
        psum_ref[...] = ps
        cnt_ref[...] = cs

    @pl.when(i > 0)
    def _():
        psum_ref[...] += ps
        cnt_ref[...] += cs


_t3 = pl.pallas_call(
    _t3_body,
    grid=(NP // RB,),
    in_specs=[
        pl.BlockSpec((RB, HP), lambda i: (i, 0)),
        pl.BlockSpec((RB, 1), lambda i: (i, 0)),
        pl.BlockSpec((HP,), lambda i: (0,)),
        pl.BlockSpec((RB, 1), lambda i: (i, 0)),
    ],
    out_specs=(
        pl.BlockSpec((G, HP), lambda i: (0, 0)),
        pl.BlockSpec((G, 1), lambda i: (0, 0)),
    ),
    out_shape=(
        jax.ShapeDtypeStruct((G, HP), _F32),
        jax.ShapeDtypeStruct((G, 1), _F32),
    ),
)


def _t4_body(psp_ref, cp_ref, psd_ref, cd_ref, w1_ref, b1_ref, w2_ref, b2_ref,
             out_ref):
    pp = psp_ref[...][:, :H] / jnp.maximum(cp_ref[...], 1.0)
    pd = psd_ref[...][:, :H] / jnp.maximum(cd_ref[...], 1.0)
    xc = jnp.concatenate([pp, pd], axis=1)
    z = jax.nn.relu(jnp.dot(xc, w1_ref[...]) + b1_ref[...][None, :])
    out_ref[...] = jnp.dot(z, w2_ref[...]) + b2_ref[0]


_t4 = pl.pallas_call(
    _t4_body,
    in_specs=[
        pl.BlockSpec((G, HP), lambda: (0, 0)),
        pl.BlockSpec((G, 1), lambda: (0, 0)),
        pl.BlockSpec((G, HP), lambda: (0, 0)),
        pl.BlockSpec((G, 1), lambda: (0, 0)),
        pl.BlockSpec((2 * H, H), lambda: (0, 0)),
        pl.BlockSpec((H,), lambda: (0,)),
        pl.BlockSpec((H, 1), lambda: (0, 0)),
        pl.BlockSpec(memory_space=pltpu.SMEM),
    ],
    out_specs=pl.BlockSpec((G, 1), lambda: (0, 0)),
    out_shape=jax.ShapeDtypeStruct((G, 1), _F32),
)


def _prep_edges(edge_index):
    pad = jnp.full((EPAD - E,), N, jnp.int32)
    src = jnp.concatenate([edge_index[0], pad]).reshape(NTILE, NCH, C)
    dst = jnp.concatenate([edge_index[1], pad]).reshape(NTILE, NCH, C)
    return src, dst


def _pad_rows(x):
    return jnp.concatenate([x, jnp.zeros((NP - N, x.shape[1]), x.dtype)])


def _pad_w(w):
    return jnp.pad(w, ((0, 0), (0, HP - w.shape[1])))


def _pad_b(b):
    return jnp.pad(b, (0, HP - b.shape[0]))


def _pad_w2(w):
    return jnp.pad(w, ((0, HP - w.shape[0]), (0, HP - w.shape[1])))


def kernel(x_p, x_d, edge_attr_p, edge_attr_d, edge_index_p, edge_index_d,
           x_p_batch, x_d_batch,
           Wp1, bp1, Wp2, bp2, Wp3, bp3,
           Wd1, bd1, Wd2, bd2, Wd3, bd3,
           Wl1, bl1, Wl2, bl2):
    del edge_attr_p, edge_attr_d

    src_p, dst_p = _prep_edges(edge_index_p)
    src_d, dst_d = _prep_edges(edge_index_d)
    src_all = jnp.stack([src_p, src_d])
    dst_all = jnp.stack([dst_p, dst_d])

    xp = _pad_rows(x_p)
    xd = _pad_rows(x_d)
    bpad = jnp.full((NP - N,), G, jnp.int32)
    batch_p = jnp.concatenate([x_p_batch, bpad]).reshape(NP, 1)
    batch_d = jnp.concatenate([x_d_batch, bpad]).reshape(NP, 1)

    deg_p, deg_d = _deg_kernel()(dst_all)
    dinv = _t0(deg_p, deg_d)
    dinv_p, dinv_d = dinv[0], dinv[1]

    y1p = _t1(xp, _pad_w(Wp1), dinv_p)
    y1d = _t1(xd, _pad_w(Wd1), dinv_d)
    s1p, s1d = _scatter_kernel()(y1p, y1d, src_all, dst_all)

    y2p = _t2(s1p, dinv_p, _pad_b(bp1), _pad_w2(Wp2))
    y2d = _t2(s1d, dinv_d, _pad_b(bd1), _pad_w2(Wd2))
    s2p, s2d = _scatter_kernel()(y2p, y2d, src_all, dst_all)

    y3p = _t2(s2p, dinv_p, _pad_b(bp2), _pad_w2(Wp3))
    y3d = _t2(s2d, dinv_d, _pad_b(bd2), _pad_w2(Wd3))
    s3p, s3d = _scatter_kernel()(y3p, y3d, src_all, dst_all)

    psp, cp = _t3(s3p, dinv_p, _pad_b(bp3), batch_p)
    psd, cd = _t3(s3d, dinv_d, _pad_b(bd3), batch_d)

    return _t4(psp, cp, psd, cd, Wl1, bl1, Wl2, bl2)
